# baseline (speedup 1.0000x reference)
import functools

import jax
import jax.numpy as jnp
from jax import lax
from jax.experimental import pallas as pl
from jax.experimental.pallas import tpu as pltpu
from jax.experimental.pallas import tpu_sc as plsc

F32 = jnp.float32

B = 2
N = 10000
N_PAD = 10240
F_IN = 256
E = 160000
E_REAL = E + N
H1, C = 8, 64
HC1 = H1 * C
H_PAD = 128
AUG = 80
TILES = 32
EC = 5376
EP = TILES * EC
W = 64
NW = EC // W
NG = NW // 3
GPW = W // 16
BN = 256
GRID_N = N_PAD // BN
ROWS_PER_TILE = N_PAD // 16


def _tc1_body(x_ref, w_ref, as_ref, ad_ref, haug_ref, at_ref, bt_ref, gm_ref):
    h = jnp.dot(x_ref[0], w_ref[...], preferred_element_type=F32)
    a_s = jnp.dot(h, as_ref[...], preferred_element_type=F32)
    a_d = jnp.dot(h, ad_ref[...], preferred_element_type=F32)
    ones = jnp.ones((BN, 1), F32)
    zeros = jnp.zeros((BN, AUG - C - 1), F32)
    for hh in range(H1):
        haug_ref[0, hh] = jnp.concatenate(
            [h[:, hh * C:(hh + 1) * C], ones, zeros], axis=1)
    at_ref[0] = a_s.T
    bt_ref[0] = a_d.T

    @pl.when(pl.program_id(1) == 0)
    def _():
        gm_ref[...] = jnp.full((1, 2, H_PAD), -3e38, F32)

    gm_ref[0, 0:1] = jnp.maximum(gm_ref[0, 0:1],
                                 jnp.max(a_s, axis=0, keepdims=True))
    gm_ref[0, 1:2] = jnp.maximum(gm_ref[0, 1:2],
                                 jnp.max(a_d, axis=0, keepdims=True))


def _tc1(x_p, W1, A1s, A1d):
    return pl.pallas_call(
        _tc1_body,
        grid=(B, GRID_N),
        in_specs=[
            pl.BlockSpec((1, BN, F_IN), lambda b, i: (b, i, 0)),
            pl.BlockSpec((F_IN, HC1), lambda b, i: (0, 0)),
            pl.BlockSpec((HC1, H_PAD), lambda b, i: (0, 0)),
            pl.BlockSpec((HC1, H_PAD), lambda b, i: (0, 0)),
        ],
        out_specs=[
            pl.BlockSpec((1, H1, BN, AUG), lambda b, i: (b, 0, i, 0)),
            pl.BlockSpec((1, H_PAD, BN), lambda b, i: (b, 0, i)),
            pl.BlockSpec((1, H_PAD, BN), lambda b, i: (b, 0, i)),
            pl.BlockSpec((1, 2, H_PAD), lambda b, i: (b, 0, 0)),
        ],
        out_shape=[
            jax.ShapeDtypeStruct((B, H1, N_PAD, AUG), F32),
            jax.ShapeDtypeStruct((B, H_PAD, N_PAD), F32),
            jax.ShapeDtypeStruct((B, H_PAD, N_PAD), F32),
            jax.ShapeDtypeStruct((B, 2, H_PAD), F32),
        ],
    )(x_p, W1, A1s, A1d)


def _tc2_body(num_ref, den_ref, e_ref, b1_ref, w2_ref, as_ref, ad_ref,
              haug_ref, at_ref, bt_ref, gm_ref):
    x = num_ref[0, 0] + num_ref[1, 0]
    d = den_ref[0, 0, :, :, 0] + den_ref[1, 0, :, :, 0]
    db = jnp.dot(d.T, e_ref[...], preferred_element_type=F32)
    x1 = jnp.maximum(x / (db + 1e-16) + b1_ref[...], 0.0)
    h2 = jnp.dot(x1, w2_ref[...], preferred_element_type=F32)
    a_s = jnp.dot(h2, as_ref[...], preferred_element_type=F32)
    a_d = jnp.dot(h2, ad_ref[...], preferred_element_type=F32)
    ones = jnp.ones((BN, 1), F32)
    zeros = jnp.zeros((BN, AUG - C - 1), F32)
    haug_ref[0, 0] = jnp.concatenate([h2, ones, zeros], axis=1)
    at_ref[0] = a_s.T
    bt_ref[0] = a_d.T

    @pl.when(pl.program_id(1) == 0)
    def _():
        gm_ref[...] = jnp.full((1, 2, H_PAD), -3e38, F32)

    gm_ref[0, 0:1] = jnp.maximum(gm_ref[0, 0:1],
                                 jnp.max(a_s, axis=0, keepdims=True))
    gm_ref[0, 1:2] = jnp.maximum(gm_ref[0, 1:2],
                                 jnp.max(a_d, axis=0, keepdims=True))


def _tc2(num1, den1, E1, b1, W2, A2s, A2d):
    return pl.pallas_call(
        _tc2_body,
        grid=(B, GRID_N),
        in_specs=[
            pl.BlockSpec((2, 1, BN, HC1), lambda b, i: (0, b, i, 0)),
            pl.BlockSpec((2, 1, H1, BN, 16), lambda b, i: (0, b, 0, i, 0)),
            pl.BlockSpec((H1, HC1), lambda b, i: (0, 0)),
            pl.BlockSpec((1, HC1), lambda b, i: (0, 0)),
            pl.BlockSpec((HC1, C), lambda b, i: (0, 0)),
            pl.BlockSpec((C, H_PAD), lambda b, i: (0, 0)),
            pl.BlockSpec((C, H_PAD), lambda b, i: (0, 0)),
        ],
        out_specs=[
            pl.BlockSpec((1, 1, BN, AUG), lambda b, i: (b, 0, i, 0)),
            pl.BlockSpec((1, H_PAD, BN), lambda b, i: (b, 0, i)),
            pl.BlockSpec((1, H_PAD, BN), lambda b, i: (b, 0, i)),
            pl.BlockSpec((1, 2, H_PAD), lambda b, i: (b, 0, 0)),
        ],
        out_shape=[
            jax.ShapeDtypeStruct((B, 1, N_PAD, AUG), F32),
            jax.ShapeDtypeStruct((B, H_PAD, N_PAD), F32),
            jax.ShapeDtypeStruct((B, H_PAD, N_PAD), F32),
            jax.ShapeDtypeStruct((B, 2, H_PAD), F32),
        ],
    )(num1, den1, E1, b1, W2, A2s, A2d)


def _tc3_body(num_ref, den_ref, b2_ref, out_ref):
    a = num_ref[0, 0, :, 0:C] + num_ref[1, 0, :, 0:C]
    d = den_ref[0, 0, 0, :, 0] + den_ref[1, 0, 0, :, 0]
    out_ref[0] = a / (d[:, None] + 1e-16) + b2_ref[...]


def _tc3(num2, den2, b2):
    return pl.pallas_call(
        _tc3_body,
        grid=(B, GRID_N),
        in_specs=[
            pl.BlockSpec((2, 1, BN, 2 * C), lambda b, i: (0, b, i, 0)),
            pl.BlockSpec((2, 1, 1, BN, 16), lambda b, i: (0, b, 0, i, 0)),
            pl.BlockSpec((1, C), lambda b, i: (0, 0)),
        ],
        out_specs=pl.BlockSpec((1, BN, C), lambda b, i: (b, i, 0)),
        out_shape=jax.ShapeDtypeStruct((B, N_PAD, C), F32),
    )(num2, den2, b2)


def _sc_edge_body(H, haug_hbm, at_hbm, bt_hbm, gm_hbm, src_hbm, dst_hbm,
                  num_hbm, den_hbm, src_v, dst_v, p_v, at_v, bt_v, gm_v,
                  rb, zb, gsem, ssem, acc_sh):
    cid = lax.axis_index("c")
    sid = lax.axis_index("s")
    wid = cid * 16 + sid

    pltpu.sync_copy(gm_hbm, gm_v)

    def zero_zb(i, _):
        for v in range(AUG // 16):
            zb[i, pl.ds(v * 16, 16)] = jnp.zeros((16,), F32)
        return _
    lax.fori_loop(0, 64, zero_zb, None)

    row0 = sid * ROWS_PER_TILE

    def zero_acc():
        for r in range(ROWS_PER_TILE // 64):
            pltpu.async_copy(zb, acc_sh.at[pl.ds(row0 + r * 64, 64)],
                             gsem.at[0])
        for r in range(ROWS_PER_TILE // 64):
            pltpu.make_async_copy(zb, acc_sh.at[pl.ds(row0 + r * 64, 64)],
                                  gsem.at[0]).wait()

    zero_acc()
    plsc.subcore_barrier()

    ebase = wid * EC

    def start_scatter(w, slot):
        pltpu.async_copy(rb.at[slot], acc_sh.at[dst_v.at[w]],
                         ssem.at[slot], add=True)

    def wait_scatter(w, slot):
        pltpu.make_async_copy(rb.at[slot], acc_sh.at[dst_v.at[w]],
                              ssem.at[slot]).wait()

    for bb in range(B):
        with jax.named_scope("edge_stage"):
            pltpu.sync_copy(src_hbm.at[bb, wid], src_v)
            pltpu.sync_copy(dst_hbm.at[bb, wid], dst_v)

        for hh in range(H):
            tbl = haug_hbm.at[bb, hh]

            def start_gather(w, slot):
                pltpu.async_copy(tbl.at[src_v.at[w]], rb.at[slot],
                                 gsem.at[slot])

            def wait_gather(w, slot):
                pltpu.make_async_copy(tbl.at[src_v.at[w]], rb.at[slot],
                                      gsem.at[slot]).wait()

            with jax.named_scope("head_stage"):
                pltpu.sync_copy(at_hbm.at[bb, hh], at_v)
                pltpu.sync_copy(bt_hbm.at[bb, hh], bt_v)

            for b in range(3):
                start_gather(b, b)

            gm0 = gm_v[bb, 0, pl.ds(0, 16)]
            gm1 = gm_v[bb, 1, pl.ds(0, 16)]
            gms = gm0[hh] + gm1[hh]
            gms = jnp.maximum(gms, 0.2 * gms)
            gmvec = lax.broadcast(gms, (16,))

            with jax.named_scope("p_compute"):
                def p_body(g, _):
                    w = g // GPW
                    col = (g % GPW) * 16
                    sv = src_v[w, pl.ds(col, 16)]
                    dv = dst_v[w, pl.ds(col, 16)]
                    a = (plsc.load_gather(at_v, [sv])
                         + plsc.load_gather(bt_v, [dv]))
                    a = jnp.maximum(a, 0.2 * a)
                    p = jnp.exp(a - gmvec)
                    gid = lax.iota(jnp.int32, 16) + (ebase + g * 16)
                    p = jnp.where(gid < E_REAL, p, 0.0)
                    p_v[w, pl.ds(col, 16)] = p
                    return _
                lax.fori_loop(0, NW * GPW, p_body, None)

            def scale_rows(w, slot):
                wvec = lax.broadcast(w, (16,))

                @plsc.parallel_loop(0, W, unroll=4)
                def _(e):
                    pvec = plsc.load_gather(
                        p_v, [wvec, lax.broadcast(e, (16,))])
                    for v in range(AUG // 16):
                        sl = pl.ds(v * 16, 16)
                        rb[slot, e, sl] = rb[slot, e, sl] * pvec

            with jax.named_scope("edge_windows"):
                def group_body(g, _):
                    s0 = (g % 2) * 3
                    q0 = 3 - s0

                    @pl.when(g < NG - 1)
                    def _():
                        for b in range(3):
                            @pl.when(g >= 1)
                            def _():
                                wait_scatter((g - 1) * 3 + b, q0 + b)
                            start_gather((g + 1) * 3 + b, q0 + b)

                    for b in range(3):
                        w = g * 3 + b
                        wait_gather(w, s0 + b)
                        scale_rows(w, s0 + b)
                        start_scatter(w, s0 + b)
                    return _
                lax.fori_loop(0, NG, group_body, None)

                for b in range(3):
                    wait_scatter((NG - 2) * 3 + b, b)
                    wait_scatter((NG - 1) * 3 + b, 3 + b)
            plsc.subcore_barrier()

            with jax.named_scope("acc_out"):
                pltpu.sync_copy(
                    acc_sh.at[pl.ds(row0, ROWS_PER_TILE), pl.ds(0, C)],
                    num_hbm.at[cid, bb, pl.ds(row0, ROWS_PER_TILE),
                               pl.ds(hh * C, C)])
                pltpu.sync_copy(
                    acc_sh.at[pl.ds(row0, ROWS_PER_TILE), pl.ds(C, 16)],
                    den_hbm.at[cid, bb, hh, pl.ds(row0, ROWS_PER_TILE)])
                if not (bb == B - 1 and hh == H - 1):
                    zero_acc()
                    plsc.subcore_barrier()


@functools.lru_cache(maxsize=None)
def _make_sc_edge(H):
    mesh = plsc.VectorSubcoreMesh(core_axis_name="c", subcore_axis_name="s",
                                  num_cores=2, num_subcores=16)
    numc = H * C if H > 1 else 2 * C
    return pl.kernel(
        functools.partial(_sc_edge_body, H),
        out_type=(jax.ShapeDtypeStruct((2, B, N_PAD, numc), F32),
                  jax.ShapeDtypeStruct((2, B, H, N_PAD, 16), F32)),
        mesh=mesh,
        scratch_types=[
            pltpu.VMEM((NW, W), jnp.int32),
            pltpu.VMEM((NW, W), jnp.int32),
            pltpu.VMEM((NW, W), F32),
            pltpu.VMEM((N_PAD,), F32),
            pltpu.VMEM((N_PAD,), F32),
            pltpu.VMEM((B, 2, H_PAD), F32),
            pltpu.VMEM((6, W, AUG), F32),
            pltpu.VMEM((64, AUG), F32),
            pltpu.SemaphoreType.DMA((6,)),
            pltpu.SemaphoreType.DMA((6,)),
            pltpu.VMEM_SHARED((N_PAD, AUG), F32),
        ],
        compiler_params=pltpu.CompilerParams(needs_layout_passes=False,
                                             use_tc_tiling_on_sc=False),
    )


def kernel(xs, edge_indexs, W1, a_src1, a_dst1, b1, W2, a_src2, a_dst2, b2):
    def block_diag_a(a, Hh):
        m = jnp.zeros((Hh * C, H_PAD), F32)
        for hh in range(Hh):
            m = m.at[hh * C:(hh + 1) * C, hh].set(a[hh])
        return m

    A1s = block_diag_a(a_src1, H1)
    A1d = block_diag_a(a_dst1, H1)
    A2s = block_diag_a(a_src2, 1)
    A2d = block_diag_a(a_dst2, 1)
    b1r = b1.reshape(1, HC1)
    b2r = b2.reshape(1, C)
    E1 = jnp.zeros((H1, HC1), F32)
    for hh in range(H1):
        E1 = E1.at[hh, hh * C:(hh + 1) * C].set(1.0)

    x_p = jnp.pad(xs, ((0, 0), (0, N_PAD - N), (0, 0)))
    loop = jnp.arange(N, dtype=edge_indexs.dtype)
    loop2 = jnp.broadcast_to(loop[None], (B, N))
    pad_n = EP - E_REAL
    pad_idx = jnp.broadcast_to(
        (jnp.arange(pad_n, dtype=edge_indexs.dtype) * 4) % N, (B, pad_n))
    src = jnp.concatenate([edge_indexs[:, 0, :], loop2, pad_idx], axis=1)
    dst = jnp.concatenate([edge_indexs[:, 1, :], loop2, pad_idx], axis=1)
    src_t = src.reshape(B, TILES, NW, W)
    dst_t = dst.reshape(B, TILES, NW, W)

    haug1, at1, bt1, gm1 = _tc1(x_p, W1, A1s, A1d)
    num1, den1 = _make_sc_edge(H1)(haug1, at1, bt1, gm1, src_t, dst_t)
    haug2, at2, bt2, gm2 = _tc2(num1, den1, E1, b1r, W2, A2s, A2d)
    num2, den2 = _make_sc_edge(1)(haug2, at2, bt2, gm2, src_t, dst_t)
    out = _tc3(num2, den2, b2r)
    return out[:, :N, :]


# speedup vs baseline: 1.0894x; 1.0894x over previous
import functools

import jax
import jax.numpy as jnp
from jax import lax
from jax.experimental import pallas as pl
from jax.experimental.pallas import tpu as pltpu
from jax.experimental.pallas import tpu_sc as plsc

F32 = jnp.float32

B = 2
N = 10000
N_PAD = 10240
F_IN = 256
E = 160000
E_REAL = E + N
H1, C = 8, 64
HC1 = H1 * C
H_PAD = 128
AUG = 80
TILES = 32
EC = 5376
EP = TILES * EC
W = 64
NW = EC // W
NG = NW // 3
GPW = W // 16
BN = 256
GRID_N = N_PAD // BN
ROWS_PER_TILE = N_PAD // 16


def _tc1_body(x_ref, w_ref, as_ref, ad_ref, haug_ref, at_ref, bt_ref, gm_ref):
    h = jnp.dot(x_ref[0], w_ref[...], preferred_element_type=F32)
    a_s = jnp.dot(h, as_ref[...], preferred_element_type=F32)
    a_d = jnp.dot(h, ad_ref[...], preferred_element_type=F32)
    ones = jnp.ones((BN, 1), F32)
    zeros = jnp.zeros((BN, AUG - C - 1), F32)
    for hh in range(H1):
        haug_ref[0, hh] = jnp.concatenate(
            [h[:, hh * C:(hh + 1) * C], ones, zeros], axis=1)
    at_ref[0] = a_s.T
    bt_ref[0] = a_d.T

    @pl.when(pl.program_id(1) == 0)
    def _():
        gm_ref[...] = jnp.full((1, 2, H_PAD), -3e38, F32)

    gm_ref[0, 0:1] = jnp.maximum(gm_ref[0, 0:1],
                                 jnp.max(a_s, axis=0, keepdims=True))
    gm_ref[0, 1:2] = jnp.maximum(gm_ref[0, 1:2],
                                 jnp.max(a_d, axis=0, keepdims=True))


def _tc1(x_p, W1, A1s, A1d):
    return pl.pallas_call(
        _tc1_body,
        grid=(B, GRID_N),
        in_specs=[
            pl.BlockSpec((1, BN, F_IN), lambda b, i: (b, i, 0)),
            pl.BlockSpec((F_IN, HC1), lambda b, i: (0, 0)),
            pl.BlockSpec((HC1, H_PAD), lambda b, i: (0, 0)),
            pl.BlockSpec((HC1, H_PAD), lambda b, i: (0, 0)),
        ],
        out_specs=[
            pl.BlockSpec((1, H1, BN, AUG), lambda b, i: (b, 0, i, 0)),
            pl.BlockSpec((1, H_PAD, BN), lambda b, i: (b, 0, i)),
            pl.BlockSpec((1, H_PAD, BN), lambda b, i: (b, 0, i)),
            pl.BlockSpec((1, 2, H_PAD), lambda b, i: (b, 0, 0)),
        ],
        out_shape=[
            jax.ShapeDtypeStruct((B, H1, N_PAD, AUG), F32),
            jax.ShapeDtypeStruct((B, H_PAD, N_PAD), F32),
            jax.ShapeDtypeStruct((B, H_PAD, N_PAD), F32),
            jax.ShapeDtypeStruct((B, 2, H_PAD), F32),
        ],
    )(x_p, W1, A1s, A1d)


def _tc2_body(acc_ref, b1_ref, w2_ref, as_ref, ad_ref,
              haug_ref, at_ref, bt_ref, gm_ref):
    cols = []
    for hh in range(H1):
        a = acc_ref[0, 0, hh] + acc_ref[1, 0, hh]
        cols.append(a[:, 0:C] / (a[:, C:C + 1] + 1e-16))
    x1 = jnp.concatenate(cols, axis=1) + b1_ref[...]
    x1 = jnp.maximum(x1, 0.0)
    h2 = jnp.dot(x1, w2_ref[...], preferred_element_type=F32)
    a_s = jnp.dot(h2, as_ref[...], preferred_element_type=F32)
    a_d = jnp.dot(h2, ad_ref[...], preferred_element_type=F32)
    ones = jnp.ones((BN, 1), F32)
    zeros = jnp.zeros((BN, AUG - C - 1), F32)
    haug_ref[0, 0] = jnp.concatenate([h2, ones, zeros], axis=1)
    at_ref[0] = a_s.T
    bt_ref[0] = a_d.T

    @pl.when(pl.program_id(1) == 0)
    def _():
        gm_ref[...] = jnp.full((1, 2, H_PAD), -3e38, F32)

    gm_ref[0, 0:1] = jnp.maximum(gm_ref[0, 0:1],
                                 jnp.max(a_s, axis=0, keepdims=True))
    gm_ref[0, 1:2] = jnp.maximum(gm_ref[0, 1:2],
                                 jnp.max(a_d, axis=0, keepdims=True))


def _tc2(acc1, b1, W2, A2s, A2d):
    return pl.pallas_call(
        _tc2_body,
        grid=(B, GRID_N),
        in_specs=[
            pl.BlockSpec((2, 1, H1, BN, AUG), lambda b, i: (0, b, 0, i, 0)),
            pl.BlockSpec((1, HC1), lambda b, i: (0, 0)),
            pl.BlockSpec((HC1, C), lambda b, i: (0, 0)),
            pl.BlockSpec((C, H_PAD), lambda b, i: (0, 0)),
            pl.BlockSpec((C, H_PAD), lambda b, i: (0, 0)),
        ],
        out_specs=[
            pl.BlockSpec((1, 1, BN, AUG), lambda b, i: (b, 0, i, 0)),
            pl.BlockSpec((1, H_PAD, BN), lambda b, i: (b, 0, i)),
            pl.BlockSpec((1, H_PAD, BN), lambda b, i: (b, 0, i)),
            pl.BlockSpec((1, 2, H_PAD), lambda b, i: (b, 0, 0)),
        ],
        out_shape=[
            jax.ShapeDtypeStruct((B, 1, N_PAD, AUG), F32),
            jax.ShapeDtypeStruct((B, H_PAD, N_PAD), F32),
            jax.ShapeDtypeStruct((B, H_PAD, N_PAD), F32),
            jax.ShapeDtypeStruct((B, 2, H_PAD), F32),
        ],
    )(acc1, b1, W2, A2s, A2d)


def _tc3_body(acc_ref, b2_ref, out_ref):
    a = acc_ref[0, 0, 0] + acc_ref[1, 0, 0]
    out_ref[0] = a[:, 0:C] / (a[:, C:C + 1] + 1e-16) + b2_ref[...]


def _tc3(acc2, b2):
    return pl.pallas_call(
        _tc3_body,
        grid=(B, GRID_N),
        in_specs=[
            pl.BlockSpec((2, 1, 1, BN, AUG), lambda b, i: (0, b, 0, i, 0)),
            pl.BlockSpec((1, C), lambda b, i: (0, 0)),
        ],
        out_specs=pl.BlockSpec((1, BN, C), lambda b, i: (b, i, 0)),
        out_shape=jax.ShapeDtypeStruct((B, N_PAD, C), F32),
    )(acc2, b2)


def _sc_edge_body(H, haug_hbm, at_hbm, bt_hbm, gm_hbm, src_hbm, dst_hbm,
                  out_hbm, src_v, dst_v, p_v, at_v, bt_v, gm_v,
                  rb, zb, gsem, ssem, tsem, acc_sh):
    cid = lax.axis_index("c")
    sid = lax.axis_index("s")
    wid = cid * 16 + sid

    pltpu.sync_copy(gm_hbm, gm_v)

    def zero_zb(i, _):
        for v in range(AUG // 16):
            zb[i, pl.ds(v * 16, 16)] = jnp.zeros((16,), F32)
        return _
    lax.fori_loop(0, 64, zero_zb, None)

    row0 = sid * ROWS_PER_TILE

    def zero_acc():
        for r in range(ROWS_PER_TILE // 64):
            pltpu.async_copy(zb, acc_sh.at[pl.ds(row0 + r * 64, 64)],
                             gsem.at[0])
        for r in range(ROWS_PER_TILE // 64):
            pltpu.make_async_copy(zb, acc_sh.at[pl.ds(row0 + r * 64, 64)],
                                  gsem.at[0]).wait()

    zero_acc()
    plsc.subcore_barrier()

    ebase = wid * EC

    def start_scatter(w, slot):
        pltpu.async_copy(rb.at[slot], acc_sh.at[dst_v.at[w]],
                         ssem.at[slot], add=True)

    def wait_scatter(w, slot):
        pltpu.make_async_copy(rb.at[slot], acc_sh.at[dst_v.at[w]],
                              ssem.at[slot]).wait()

    def stage_tables(bb, hh):
        pltpu.async_copy(at_hbm.at[bb, hh], at_v, tsem.at[0])
        pltpu.async_copy(bt_hbm.at[bb, hh], bt_v, tsem.at[1])

    def wait_tables(bb, hh):
        pltpu.make_async_copy(at_hbm.at[bb, hh], at_v, tsem.at[0]).wait()
        pltpu.make_async_copy(bt_hbm.at[bb, hh], bt_v, tsem.at[1]).wait()

    stage_tables(0, 0)

    for bb in range(B):
        with jax.named_scope("edge_stage"):
            pltpu.sync_copy(src_hbm.at[bb, wid], src_v)
            pltpu.sync_copy(dst_hbm.at[bb, wid], dst_v)

        for hh in range(H):
            tbl = haug_hbm.at[bb, hh]

            def start_gather(w, slot):
                pltpu.async_copy(tbl.at[src_v.at[w]], rb.at[slot],
                                 gsem.at[slot])

            def wait_gather(w, slot):
                pltpu.make_async_copy(tbl.at[src_v.at[w]], rb.at[slot],
                                      gsem.at[slot]).wait()

            with jax.named_scope("head_stage"):
                wait_tables(bb, hh)

            for b in range(3):
                start_gather(b, b)

            gm0 = gm_v[bb, 0, pl.ds(0, 16)]
            gm1 = gm_v[bb, 1, pl.ds(0, 16)]
            gms = gm0[hh] + gm1[hh]
            gms = jnp.maximum(gms, 0.2 * gms)
            gmvec = lax.broadcast(gms, (16,))

            with jax.named_scope("p_compute"):
                def p_body(g, _):
                    w = g // GPW
                    col = (g % GPW) * 16
                    sv = src_v[w, pl.ds(col, 16)]
                    dv = dst_v[w, pl.ds(col, 16)]
                    a = (plsc.load_gather(at_v, [sv])
                         + plsc.load_gather(bt_v, [dv]))
                    a = jnp.maximum(a, 0.2 * a)
                    p = jnp.exp(a - gmvec)
                    gid = lax.iota(jnp.int32, 16) + (ebase + g * 16)
                    p = jnp.where(gid < E_REAL, p, 0.0)
                    p_v[w, pl.ds(col, 16)] = p
                    return _
                lax.fori_loop(0, NW * GPW, p_body, None)

            if hh < H - 1:
                stage_tables(bb, hh + 1)
            elif bb < B - 1:
                stage_tables(bb + 1, 0)

            def scale_rows(w, slot):
                wvec = lax.broadcast(w, (16,))

                @plsc.parallel_loop(0, W, unroll=4)
                def _(e):
                    pvec = plsc.load_gather(
                        p_v, [wvec, lax.broadcast(e, (16,))])
                    for v in range(AUG // 16):
                        sl = pl.ds(v * 16, 16)
                        rb[slot, e, sl] = rb[slot, e, sl] * pvec

            with jax.named_scope("edge_windows"):
                def group_body(g, _):
                    s0 = (g % 2) * 3
                    q0 = 3 - s0

                    @pl.when(g < NG - 1)
                    def _():
                        for b in range(3):
                            @pl.when(g >= 1)
                            def _():
                                wait_scatter((g - 1) * 3 + b, q0 + b)
                            start_gather((g + 1) * 3 + b, q0 + b)

                    for b in range(3):
                        w = g * 3 + b
                        wait_gather(w, s0 + b)
                        scale_rows(w, s0 + b)
                        start_scatter(w, s0 + b)
                    return _
                lax.fori_loop(0, NG, group_body, None)

                for b in range(3):
                    wait_scatter((NG - 2) * 3 + b, b)
                    wait_scatter((NG - 1) * 3 + b, 3 + b)
            plsc.subcore_barrier()

            with jax.named_scope("acc_out"):
                pltpu.sync_copy(
                    acc_sh.at[pl.ds(row0, ROWS_PER_TILE)],
                    out_hbm.at[cid, bb, hh, pl.ds(row0, ROWS_PER_TILE)])
                if not (bb == B - 1 and hh == H - 1):
                    zero_acc()
                    plsc.subcore_barrier()


@functools.lru_cache(maxsize=None)
def _make_sc_edge(H):
    mesh = plsc.VectorSubcoreMesh(core_axis_name="c", subcore_axis_name="s",
                                  num_cores=2, num_subcores=16)
    return pl.kernel(
        functools.partial(_sc_edge_body, H),
        out_type=jax.ShapeDtypeStruct((2, B, H, N_PAD, AUG), F32),
        mesh=mesh,
        scratch_types=[
            pltpu.VMEM((NW, W), jnp.int32),
            pltpu.VMEM((NW, W), jnp.int32),
            pltpu.VMEM((NW, W), F32),
            pltpu.VMEM((N_PAD,), F32),
            pltpu.VMEM((N_PAD,), F32),
            pltpu.VMEM((B, 2, H_PAD), F32),
            pltpu.VMEM((6, W, AUG), F32),
            pltpu.VMEM((64, AUG), F32),
            pltpu.SemaphoreType.DMA((6,)),
            pltpu.SemaphoreType.DMA((6,)),
            pltpu.SemaphoreType.DMA((2,)),
            pltpu.VMEM_SHARED((N_PAD, AUG), F32),
        ],
        compiler_params=pltpu.CompilerParams(needs_layout_passes=False,
                                             use_tc_tiling_on_sc=False),
    )


def kernel(xs, edge_indexs, W1, a_src1, a_dst1, b1, W2, a_src2, a_dst2, b2):
    def block_diag_a(a, Hh):
        m = jnp.zeros((Hh * C, H_PAD), F32)
        for hh in range(Hh):
            m = m.at[hh * C:(hh + 1) * C, hh].set(a[hh])
        return m

    A1s = block_diag_a(a_src1, H1)
    A1d = block_diag_a(a_dst1, H1)
    A2s = block_diag_a(a_src2, 1)
    A2d = block_diag_a(a_dst2, 1)
    b1r = b1.reshape(1, HC1)
    b2r = b2.reshape(1, C)

    x_p = jnp.pad(xs, ((0, 0), (0, N_PAD - N), (0, 0)))
    loop = jnp.arange(N, dtype=edge_indexs.dtype)
    loop2 = jnp.broadcast_to(loop[None], (B, N))
    pad_n = EP - E_REAL
    pad_idx = jnp.broadcast_to(
        (jnp.arange(pad_n, dtype=edge_indexs.dtype) * 4) % N, (B, pad_n))
    src = jnp.concatenate([edge_indexs[:, 0, :], loop2, pad_idx], axis=1)
    dst = jnp.concatenate([edge_indexs[:, 1, :], loop2, pad_idx], axis=1)
    src_t = src.reshape(B, TILES, NW, W)
    dst_t = dst.reshape(B, TILES, NW, W)

    haug1, at1, bt1, gm1 = _tc1(x_p, W1, A1s, A1d)
    acc1 = _make_sc_edge(H1)(haug1, at1, bt1, gm1, src_t, dst_t)
    haug2, at2, bt2, gm2 = _tc2(acc1, b1r, W2, A2s, A2d)
    acc2 = _make_sc_edge(1)(haug2, at2, bt2, gm2, src_t, dst_t)
    out = _tc3(acc2, b2r)
    return out[:, :N, :]


# speedup vs baseline: 1.2833x; 1.1780x over previous
import functools

import jax
import jax.numpy as jnp
from jax import lax
from jax.experimental import pallas as pl
from jax.experimental.pallas import tpu as pltpu
from jax.experimental.pallas import tpu_sc as plsc

F32 = jnp.float32

B = 2
N = 10000
N_PAD = 10240
F_IN = 256
E = 160000
E_REAL = E + N
H1, C = 8, 64
HC1 = H1 * C
H_PAD = 128
TILES = 32
EC = 5376
EP = TILES * EC
W = 64
NW = EC // W
NG = NW // 3
GPW = W // 16
BN = 256
GRID_N = N_PAD // BN
ROWS_PER_TILE = N_PAD // 16


def _tc1_body(x_ref, w_ref, as_ref, ad_ref, haug_ref, at_ref, bt_ref, gm_ref):
    h = jnp.dot(x_ref[0], w_ref[...], preferred_element_type=F32)
    a_s = jnp.dot(h, as_ref[...], preferred_element_type=F32)
    a_d = jnp.dot(h, ad_ref[...], preferred_element_type=F32)
    for hh in range(H1):
        haug_ref[0, hh] = h[:, hh * C:(hh + 1) * C]
    at_ref[0] = a_s.T
    bt_ref[0] = a_d.T

    @pl.when(pl.program_id(1) == 0)
    def _():
        gm_ref[...] = jnp.full((1, 2, H_PAD), -3e38, F32)

    gm_ref[0, 0:1] = jnp.maximum(gm_ref[0, 0:1],
                                 jnp.max(a_s, axis=0, keepdims=True))
    gm_ref[0, 1:2] = jnp.maximum(gm_ref[0, 1:2],
                                 jnp.max(a_d, axis=0, keepdims=True))


def _tc1(x_p, W1, A1s, A1d):
    return pl.pallas_call(
        _tc1_body,
        grid=(B, GRID_N),
        in_specs=[
            pl.BlockSpec((1, BN, F_IN), lambda b, i: (b, i, 0)),
            pl.BlockSpec((F_IN, HC1), lambda b, i: (0, 0)),
            pl.BlockSpec((HC1, H_PAD), lambda b, i: (0, 0)),
            pl.BlockSpec((HC1, H_PAD), lambda b, i: (0, 0)),
        ],
        out_specs=[
            pl.BlockSpec((1, H1, BN, C), lambda b, i: (b, 0, i, 0)),
            pl.BlockSpec((1, H_PAD, BN), lambda b, i: (b, 0, i)),
            pl.BlockSpec((1, H_PAD, BN), lambda b, i: (b, 0, i)),
            pl.BlockSpec((1, 2, H_PAD), lambda b, i: (b, 0, 0)),
        ],
        out_shape=[
            jax.ShapeDtypeStruct((B, H1, N_PAD, C), F32),
            jax.ShapeDtypeStruct((B, H_PAD, N_PAD), F32),
            jax.ShapeDtypeStruct((B, H_PAD, N_PAD), F32),
            jax.ShapeDtypeStruct((B, 2, H_PAD), F32),
        ],
    )(x_p, W1, A1s, A1d)


def _tc2_body(num_ref, den_ref, e_ref, b1_ref, w2_ref, as_ref, ad_ref,
              haug_ref, at_ref, bt_ref, gm_ref):
    x = num_ref[0, 0] + num_ref[1, 0]
    d = den_ref[0, 0] + den_ref[1, 0]
    db = jnp.dot(d.T, e_ref[...], preferred_element_type=F32)
    x1 = jnp.maximum(x / (db + 1e-16) + b1_ref[...], 0.0)
    h2 = jnp.dot(x1, w2_ref[...], preferred_element_type=F32)
    a_s = jnp.dot(h2, as_ref[...], preferred_element_type=F32)
    a_d = jnp.dot(h2, ad_ref[...], preferred_element_type=F32)
    haug_ref[0, 0] = h2
    at_ref[0] = a_s.T
    bt_ref[0] = a_d.T

    @pl.when(pl.program_id(1) == 0)
    def _():
        gm_ref[...] = jnp.full((1, 2, H_PAD), -3e38, F32)

    gm_ref[0, 0:1] = jnp.maximum(gm_ref[0, 0:1],
                                 jnp.max(a_s, axis=0, keepdims=True))
    gm_ref[0, 1:2] = jnp.maximum(gm_ref[0, 1:2],
                                 jnp.max(a_d, axis=0, keepdims=True))


def _tc2(num1, den1, E1, b1, W2, A2s, A2d):
    return pl.pallas_call(
        _tc2_body,
        grid=(B, GRID_N),
        in_specs=[
            pl.BlockSpec((2, 1, BN, HC1), lambda b, i: (0, b, i, 0)),
            pl.BlockSpec((2, 1, H1, BN), lambda b, i: (0, b, 0, i)),
            pl.BlockSpec((H1, HC1), lambda b, i: (0, 0)),
            pl.BlockSpec((1, HC1), lambda b, i: (0, 0)),
            pl.BlockSpec((HC1, C), lambda b, i: (0, 0)),
            pl.BlockSpec((C, H_PAD), lambda b, i: (0, 0)),
            pl.BlockSpec((C, H_PAD), lambda b, i: (0, 0)),
        ],
        out_specs=[
            pl.BlockSpec((1, 1, BN, C), lambda b, i: (b, 0, i, 0)),
            pl.BlockSpec((1, H_PAD, BN), lambda b, i: (b, 0, i)),
            pl.BlockSpec((1, H_PAD, BN), lambda b, i: (b, 0, i)),
            pl.BlockSpec((1, 2, H_PAD), lambda b, i: (b, 0, 0)),
        ],
        out_shape=[
            jax.ShapeDtypeStruct((B, 1, N_PAD, C), F32),
            jax.ShapeDtypeStruct((B, H_PAD, N_PAD), F32),
            jax.ShapeDtypeStruct((B, H_PAD, N_PAD), F32),
            jax.ShapeDtypeStruct((B, 2, H_PAD), F32),
        ],
    )(num1, den1, E1, b1, W2, A2s, A2d)


def _tc3_body(num_ref, den_ref, b2_ref, out_ref):
    a = num_ref[0, 0, :, 0:C] + num_ref[1, 0, :, 0:C]
    d = den_ref[0, 0, 0] + den_ref[1, 0, 0]
    out_ref[0] = a / (d[:, None] + 1e-16) + b2_ref[...]


def _tc3(num2, den2, b2):
    return pl.pallas_call(
        _tc3_body,
        grid=(B, GRID_N),
        in_specs=[
            pl.BlockSpec((2, 1, BN, 2 * C), lambda b, i: (0, b, i, 0)),
            pl.BlockSpec((2, 1, 1, BN), lambda b, i: (0, b, 0, i)),
            pl.BlockSpec((1, C), lambda b, i: (0, 0)),
        ],
        out_specs=pl.BlockSpec((1, BN, C), lambda b, i: (b, i, 0)),
        out_shape=jax.ShapeDtypeStruct((B, N_PAD, C), F32),
    )(num2, den2, b2)


def _sc_edge_body(H, haug_hbm, at_hbm, bt_hbm, gm_hbm, src_hbm, dst_hbm,
                  num_hbm, den_hbm, src_v, dst_v, p_v, at_v, bt_v, gm_v,
                  rb, zb, zd, gsem, ssem, dsem, tsem, acc_sh, den_sh):
    cid = lax.axis_index("c")
    sid = lax.axis_index("s")
    wid = cid * 16 + sid

    pltpu.sync_copy(gm_hbm, gm_v)

    def zero_zb(i, _):
        for v in range(C // 16):
            zb[i, pl.ds(v * 16, 16)] = jnp.zeros((16,), F32)
        return _
    lax.fori_loop(0, 64, zero_zb, None)

    def zero_zd(i, _):
        zd[pl.ds(i * 16, 16)] = jnp.zeros((16,), F32)
        return _
    lax.fori_loop(0, ROWS_PER_TILE // 16, zero_zd, None)

    row0 = sid * ROWS_PER_TILE

    def zero_acc():
        for r in range(ROWS_PER_TILE // 64):
            pltpu.async_copy(zb, acc_sh.at[pl.ds(row0 + r * 64, 64)],
                             gsem.at[0])
        pltpu.async_copy(zd, den_sh.at[pl.ds(row0, ROWS_PER_TILE)],
                         gsem.at[1])
        for r in range(ROWS_PER_TILE // 64):
            pltpu.make_async_copy(zb, acc_sh.at[pl.ds(row0 + r * 64, 64)],
                                  gsem.at[0]).wait()
        pltpu.make_async_copy(zd, den_sh.at[pl.ds(row0, ROWS_PER_TILE)],
                              gsem.at[1]).wait()

    zero_acc()
    plsc.subcore_barrier()

    ebase = wid * EC

    def start_scatter(w, slot):
        pltpu.async_copy(rb.at[slot], acc_sh.at[dst_v.at[w]],
                         ssem.at[slot], add=True)
        pltpu.async_copy(p_v.at[w], den_sh.at[dst_v.at[w]],
                         dsem.at[slot], add=True)

    def wait_scatter(w, slot):
        pltpu.make_async_copy(rb.at[slot], acc_sh.at[dst_v.at[w]],
                              ssem.at[slot]).wait()
        pltpu.make_async_copy(p_v.at[w], den_sh.at[dst_v.at[w]],
                              dsem.at[slot]).wait()

    def stage_tables(bb, hh):
        pltpu.async_copy(at_hbm.at[bb, hh], at_v, tsem.at[0])
        pltpu.async_copy(bt_hbm.at[bb, hh], bt_v, tsem.at[1])

    def wait_tables(bb, hh):
        pltpu.make_async_copy(at_hbm.at[bb, hh], at_v, tsem.at[0]).wait()
        pltpu.make_async_copy(bt_hbm.at[bb, hh], bt_v, tsem.at[1]).wait()

    stage_tables(0, 0)

    for bb in range(B):
        with jax.named_scope("edge_stage"):
            pltpu.sync_copy(src_hbm.at[bb, wid], src_v)
            pltpu.sync_copy(dst_hbm.at[bb, wid], dst_v)

        for hh in range(H):
            tbl = haug_hbm.at[bb, hh]

            def start_gather(w, slot):
                pltpu.async_copy(tbl.at[src_v.at[w]], rb.at[slot],
                                 gsem.at[slot])

            def wait_gather(w, slot):
                pltpu.make_async_copy(tbl.at[src_v.at[w]], rb.at[slot],
                                      gsem.at[slot]).wait()

            with jax.named_scope("head_stage"):
                wait_tables(bb, hh)

            for b in range(3):
                start_gather(b, b)

            gm0 = gm_v[bb, 0, pl.ds(0, 16)]
            gm1 = gm_v[bb, 1, pl.ds(0, 16)]
            gms = gm0[hh] + gm1[hh]
            gms = jnp.maximum(gms, 0.2 * gms)
            gmvec = lax.broadcast(gms, (16,))

            with jax.named_scope("p_compute"):
                def p_body(g, _):
                    w = g // GPW
                    col = (g % GPW) * 16
                    sv = src_v[w, pl.ds(col, 16)]
                    dv = dst_v[w, pl.ds(col, 16)]
                    a = (plsc.load_gather(at_v, [sv])
                         + plsc.load_gather(bt_v, [dv]))
                    a = jnp.maximum(a, 0.2 * a)
                    p = jnp.exp(a - gmvec)
                    gid = lax.iota(jnp.int32, 16) + (ebase + g * 16)
                    p = jnp.where(gid < E_REAL, p, 0.0)
                    p_v[w, pl.ds(col, 16)] = p
                    return _
                lax.fori_loop(0, NW * GPW, p_body, None)

            if hh < H - 1:
                stage_tables(bb, hh + 1)
            elif bb < B - 1:
                stage_tables(bb + 1, 0)

            def scale_rows(w, slot):
                wvec = lax.broadcast(w, (16,))

                @plsc.parallel_loop(0, W, unroll=4)
                def _(e):
                    pvec = plsc.load_gather(
                        p_v, [wvec, lax.broadcast(e, (16,))])
                    for v in range(C // 16):
                        sl = pl.ds(v * 16, 16)
                        rb[slot, e, sl] = rb[slot, e, sl] * pvec

            with jax.named_scope("edge_windows"):
                def group_body(g, _):
                    s0 = (g % 2) * 3
                    q0 = 3 - s0

                    @pl.when(g < NG - 1)
                    def _():
                        for b in range(3):
                            @pl.when(g >= 1)
                            def _():
                                wait_scatter((g - 1) * 3 + b, q0 + b)
                            start_gather((g + 1) * 3 + b, q0 + b)

                    for b in range(3):
                        w = g * 3 + b
                        wait_gather(w, s0 + b)
                        scale_rows(w, s0 + b)
                        start_scatter(w, s0 + b)
                    return _
                lax.fori_loop(0, NG, group_body, None)

                for b in range(3):
                    wait_scatter((NG - 2) * 3 + b, b)
                    wait_scatter((NG - 1) * 3 + b, 3 + b)
            plsc.subcore_barrier()

            with jax.named_scope("acc_out"):
                pltpu.sync_copy(
                    acc_sh.at[pl.ds(row0, ROWS_PER_TILE)],
                    num_hbm.at[cid, bb, pl.ds(row0, ROWS_PER_TILE),
                               pl.ds(hh * C, C)])
                pltpu.sync_copy(
                    den_sh.at[pl.ds(row0, ROWS_PER_TILE)],
                    den_hbm.at[cid, bb, hh, pl.ds(row0, ROWS_PER_TILE)])
                if not (bb == B - 1 and hh == H - 1):
                    zero_acc()
                    plsc.subcore_barrier()


@functools.lru_cache(maxsize=None)
def _make_sc_edge(H):
    mesh = plsc.VectorSubcoreMesh(core_axis_name="c", subcore_axis_name="s",
                                  num_cores=2, num_subcores=16)
    numc = H * C if H > 1 else 2 * C
    return pl.kernel(
        functools.partial(_sc_edge_body, H),
        out_type=(jax.ShapeDtypeStruct((2, B, N_PAD, numc), F32),
                  jax.ShapeDtypeStruct((2, B, H, N_PAD), F32)),
        mesh=mesh,
        scratch_types=[
            pltpu.VMEM((NW, W), jnp.int32),
            pltpu.VMEM((NW, W), jnp.int32),
            pltpu.VMEM((NW, W), F32),
            pltpu.VMEM((N_PAD,), F32),
            pltpu.VMEM((N_PAD,), F32),
            pltpu.VMEM((B, 2, H_PAD), F32),
            pltpu.VMEM((6, W, C), F32),
            pltpu.VMEM((64, C), F32),
            pltpu.VMEM((ROWS_PER_TILE,), F32),
            pltpu.SemaphoreType.DMA((6,)),
            pltpu.SemaphoreType.DMA((6,)),
            pltpu.SemaphoreType.DMA((6,)),
            pltpu.SemaphoreType.DMA((2,)),
            pltpu.VMEM_SHARED((N_PAD, C), F32),
            pltpu.VMEM_SHARED((N_PAD,), F32),
        ],
        compiler_params=pltpu.CompilerParams(needs_layout_passes=False,
                                             use_tc_tiling_on_sc=False),
    )


def kernel(xs, edge_indexs, W1, a_src1, a_dst1, b1, W2, a_src2, a_dst2, b2):
    def block_diag_a(a, Hh):
        m = jnp.zeros((Hh * C, H_PAD), F32)
        for hh in range(Hh):
            m = m.at[hh * C:(hh + 1) * C, hh].set(a[hh])
        return m

    A1s = block_diag_a(a_src1, H1)
    A1d = block_diag_a(a_dst1, H1)
    A2s = block_diag_a(a_src2, 1)
    A2d = block_diag_a(a_dst2, 1)
    b1r = b1.reshape(1, HC1)
    b2r = b2.reshape(1, C)
    E1 = jnp.zeros((H1, HC1), F32)
    for hh in range(H1):
        E1 = E1.at[hh, hh * C:(hh + 1) * C].set(1.0)

    x_p = jnp.pad(xs, ((0, 0), (0, N_PAD - N), (0, 0)))
    loop = jnp.arange(N, dtype=edge_indexs.dtype)
    loop2 = jnp.broadcast_to(loop[None], (B, N))
    pad_n = EP - E_REAL
    pad_idx = jnp.broadcast_to(
        (jnp.arange(pad_n, dtype=edge_indexs.dtype) * 4) % N, (B, pad_n))
    src = jnp.concatenate([edge_indexs[:, 0, :], loop2, pad_idx], axis=1)
    dst = jnp.concatenate([edge_indexs[:, 1, :], loop2, pad_idx], axis=1)
    src_t = src.reshape(B, TILES, NW, W)
    dst_t = dst.reshape(B, TILES, NW, W)

    haug1, at1, bt1, gm1 = _tc1(x_p, W1, A1s, A1d)
    num1, den1 = _make_sc_edge(H1)(haug1, at1, bt1, gm1, src_t, dst_t)
    haug2, at2, bt2, gm2 = _tc2(num1, den1, E1, b1r, W2, A2s, A2d)
    num2, den2 = _make_sc_edge(1)(haug2, at2, bt2, gm2, src_t, dst_t)
    out = _tc3(num2, den2, b2r)
    return out[:, :N, :]
